# baseline (speedup 1.0000x reference)
import functools

import jax
import jax.numpy as jnp
from jax import lax
from jax.experimental import pallas as pl
from jax.experimental.pallas import tpu as pltpu
from jax.experimental.pallas import tpu_sc as plsc

EMB_DIM = 32
NUM_CORES = 2
NUM_SUBCORES = 16
NW = NUM_CORES * NUM_SUBCORES
NQ = 250112
RB = 128
_GRID = NQ // RB


def _relayout_body(i0, i1, i2, i3, out_ref):
    for b, r in enumerate((i0, i1, i2, i3)):
        out_ref[:, 32 * b : 32 * (b + 1)] = r[...].T


def _relayout(tbl_t):
    spec = lambda b: pl.BlockSpec((EMB_DIM, RB), lambda i, b=b: (0, i + b * _GRID))
    return pl.pallas_call(
        _relayout_body,
        grid=(_GRID,),
        in_specs=[spec(0), spec(1), spec(2), spec(3)],
        out_specs=pl.BlockSpec((RB, 128), lambda i: (i, 0)),
        out_shape=jax.ShapeDtypeStruct((NQ, 128), jnp.float32),
    )(tbl_t, tbl_t, tbl_t, tbl_t)


@functools.lru_cache(maxsize=None)
def _make_sc_gather(n_fields):
    mesh = plsc.VectorSubcoreMesh(core_axis_name="c", subcore_axis_name="s")

    @functools.partial(
        pl.kernel,
        mesh=mesh,
        out_type=jax.ShapeDtypeStruct((n_fields, 4, NW, 8, 128), jnp.float32),
        scratch_types=[
            pltpu.VMEM((n_fields, 128), jnp.int32),
            pltpu.VMEM((n_fields, 128), jnp.int32),
            pltpu.VMEM((128, 128), jnp.float32),
            pltpu.VMEM((128, 128), jnp.float32),
            pltpu.VMEM((4, 8, 128), jnp.float32),
            pltpu.SemaphoreType.DMA,
            pltpu.SemaphoreType.DMA,
        ],
        compiler_params=pltpu.CompilerParams(
            use_tc_tiling_on_sc=False, needs_layout_passes=False
        ),
    )
    def k(tblq_hbm, qidx_hbm, sub_hbm, out_hbm, qidx_v, sub_v, g0, g1, ot, s0, s1):
        wid = lax.axis_index("s") * NUM_CORES + lax.axis_index("c")
        pltpu.sync_copy(qidx_hbm.at[wid], qidx_v)
        pltpu.sync_copy(sub_hbm.at[wid], sub_v)
        riota = lax.iota(jnp.int32, 16)

        def process(f, g):
            for grp in range(8):
                rows = riota + (16 * grp)
                cols = sub_v[f, pl.ds(16 * grp, 16)]
                for j in range(EMB_DIM):
                    ot[j // 8, j % 8, pl.ds(16 * grp, 16)] = plsc.load_gather(
                        g, [rows, cols + j]
                    )
            for jk in range(4):
                pltpu.sync_copy(ot.at[jk], out_hbm.at[f, jk, wid])

        pltpu.async_copy(tblq_hbm.at[qidx_v.at[0]], g0, s0)
        pltpu.async_copy(tblq_hbm.at[qidx_v.at[1]], g1, s1)

        def body(fp, carry):
            f0 = 2 * fp
            pltpu.make_async_copy(tblq_hbm.at[qidx_v.at[f0]], g0, s0).wait()
            process(f0, g0)
            pltpu.async_copy(
                tblq_hbm.at[qidx_v.at[jnp.minimum(f0 + 2, n_fields - 1)]], g0, s0
            )
            f1 = f0 + 1
            pltpu.make_async_copy(tblq_hbm.at[qidx_v.at[f1]], g1, s1).wait()
            process(f1, g1)
            pltpu.async_copy(
                tblq_hbm.at[qidx_v.at[jnp.minimum(f1 + 2, n_fields - 1)]], g1, s1
            )
            return carry

        lax.fori_loop(0, n_fields // 2, body, 0)
        pltpu.make_async_copy(tblq_hbm.at[qidx_v.at[n_fields - 1]], g0, s0).wait()
        pltpu.make_async_copy(tblq_hbm.at[qidx_v.at[n_fields - 1]], g1, s1).wait()

    return k


def kernel(x, table):
    b, f = x.shape
    xf = x.astype(jnp.int32)
    tblq = _relayout(table.T)
    q = (xf % NQ).reshape(NW, b // NW, f).transpose(0, 2, 1)
    sub = ((xf // NQ) * EMB_DIM).reshape(NW, b // NW, f).transpose(0, 2, 1)
    out5 = _make_sc_gather(f)(tblq, q, sub)
    return out5.transpose(2, 4, 0, 1, 3).reshape(b, f, EMB_DIM)


# speedup vs baseline: 2.3716x; 2.3716x over previous
import functools

import jax
import jax.numpy as jnp
from jax import lax
from jax.experimental import pallas as pl
from jax.experimental.pallas import tpu as pltpu
from jax.experimental.pallas import tpu_sc as plsc

EMB_DIM = 32
NUM_CORES = 2
NUM_SUBCORES = 16
NW = NUM_CORES * NUM_SUBCORES
NQ = 250368
RB = 512
_GRID = NQ // RB


def _relayout_body(i0, i1, i2, i3, out_ref):
    eye = jnp.eye(EMB_DIM, dtype=jnp.float32)
    for b, r in enumerate((i0, i1, i2, i3)):
        out_ref[:, 32 * b : 32 * (b + 1)] = lax.dot_general(
            r[...],
            eye,
            ((( 0,), (0,)), ((), ())),
            preferred_element_type=jnp.float32,
        )


def _relayout(tbl_t):
    last = (tbl_t.shape[1] - 1) // RB
    spec = lambda b: pl.BlockSpec(
        (EMB_DIM, RB), lambda i, b=b: (0, jnp.minimum(i + b * _GRID, last))
    )
    return pl.pallas_call(
        _relayout_body,
        grid=(_GRID,),
        in_specs=[spec(0), spec(1), spec(2), spec(3)],
        out_specs=pl.BlockSpec((RB, 128), lambda i: (i, 0)),
        out_shape=jax.ShapeDtypeStruct((NQ, 128), jnp.float32),
    )(tbl_t, tbl_t, tbl_t, tbl_t)


@functools.lru_cache(maxsize=None)
def _make_sc_gather(n_fields):
    mesh = plsc.VectorSubcoreMesh(core_axis_name="c", subcore_axis_name="s")

    @functools.partial(
        pl.kernel,
        mesh=mesh,
        out_type=jax.ShapeDtypeStruct((n_fields, 4, NW, 8, 128), jnp.float32),
        scratch_types=[
            pltpu.VMEM((n_fields, 128), jnp.int32),
            pltpu.VMEM((n_fields, 128), jnp.int32),
            pltpu.VMEM((128, 128), jnp.float32),
            pltpu.VMEM((128, 128), jnp.float32),
            pltpu.VMEM((4, 8, 128), jnp.float32),
            pltpu.SemaphoreType.DMA,
            pltpu.SemaphoreType.DMA,
        ],
        compiler_params=pltpu.CompilerParams(
            use_tc_tiling_on_sc=False, needs_layout_passes=False
        ),
    )
    def k(tblq_hbm, qidx_hbm, sub_hbm, out_hbm, qidx_v, sub_v, g0, g1, ot, s0, s1):
        wid = lax.axis_index("s") * NUM_CORES + lax.axis_index("c")
        pltpu.sync_copy(qidx_hbm.at[wid], qidx_v)
        pltpu.sync_copy(sub_hbm.at[wid], sub_v)
        riota = lax.iota(jnp.int32, 16)

        def process(f, g):
            for grp in range(8):
                rows = riota + (16 * grp)
                cols = sub_v[f, pl.ds(16 * grp, 16)]
                for j in range(EMB_DIM):
                    ot[j // 8, j % 8, pl.ds(16 * grp, 16)] = plsc.load_gather(
                        g, [rows, cols + j]
                    )
            for jk in range(4):
                pltpu.sync_copy(ot.at[jk], out_hbm.at[f, jk, wid])

        pltpu.async_copy(tblq_hbm.at[qidx_v.at[0]], g0, s0)
        pltpu.async_copy(tblq_hbm.at[qidx_v.at[1]], g1, s1)

        def body(fp, carry):
            f0 = 2 * fp
            pltpu.make_async_copy(tblq_hbm.at[qidx_v.at[f0]], g0, s0).wait()
            process(f0, g0)
            pltpu.async_copy(
                tblq_hbm.at[qidx_v.at[jnp.minimum(f0 + 2, n_fields - 1)]], g0, s0
            )
            f1 = f0 + 1
            pltpu.make_async_copy(tblq_hbm.at[qidx_v.at[f1]], g1, s1).wait()
            process(f1, g1)
            pltpu.async_copy(
                tblq_hbm.at[qidx_v.at[jnp.minimum(f1 + 2, n_fields - 1)]], g1, s1
            )
            return carry

        lax.fori_loop(0, n_fields // 2, body, 0)
        pltpu.make_async_copy(tblq_hbm.at[qidx_v.at[n_fields - 1]], g0, s0).wait()
        pltpu.make_async_copy(tblq_hbm.at[qidx_v.at[n_fields - 1]], g1, s1).wait()

    return k


def kernel(x, table):
    b, f = x.shape
    xf = x.astype(jnp.int32)
    tblq = _relayout(table.T)
    q = (xf % NQ).reshape(NW, b // NW, f).transpose(0, 2, 1)
    sub = ((xf // NQ) * EMB_DIM).reshape(NW, b // NW, f).transpose(0, 2, 1)
    out5 = _make_sc_gather(f)(tblq, q, sub)
    return out5.transpose(2, 4, 0, 1, 3).reshape(b, f, EMB_DIM)


# speedup vs baseline: 3.9358x; 1.6596x over previous
import functools

import jax
import jax.numpy as jnp
from jax import lax
from jax.experimental import pallas as pl
from jax.experimental.pallas import tpu as pltpu
from jax.experimental.pallas import tpu_sc as plsc

EMB_DIM = 32
NUM_CORES = 2
NUM_SUBCORES = 16
NW = NUM_CORES * NUM_SUBCORES
NQ = 250880
RB = 1024
_GRID = NQ // RB


def _relayout_body(i0, i1, i2, i3, out_ref):
    cat = jnp.concatenate([i0[...], i1[...], i2[...], i3[...]], axis=0)
    out_ref[...] = lax.dot_general(
        cat,
        jnp.eye(128, dtype=jnp.float32),
        (((0,), (0,)), ((), ())),
        preferred_element_type=jnp.float32,
    )


def _relayout(tbl_t):
    last = (tbl_t.shape[1] - 1) // RB
    spec = lambda b: pl.BlockSpec(
        (EMB_DIM, RB), lambda i, b=b: (0, jnp.minimum(i + b * _GRID, last))
    )
    return pl.pallas_call(
        _relayout_body,
        grid=(_GRID,),
        in_specs=[spec(0), spec(1), spec(2), spec(3)],
        out_specs=pl.BlockSpec((RB, 128), lambda i: (i, 0)),
        out_shape=jax.ShapeDtypeStruct((NQ, 128), jnp.float32),
    )(tbl_t, tbl_t, tbl_t, tbl_t)


@functools.lru_cache(maxsize=None)
def _make_sc_gather(n_fields):
    mesh = plsc.VectorSubcoreMesh(core_axis_name="c", subcore_axis_name="s")

    @functools.partial(
        pl.kernel,
        mesh=mesh,
        out_type=jax.ShapeDtypeStruct((n_fields, 4, NW, 8, 128), jnp.float32),
        scratch_types=[
            pltpu.VMEM((n_fields, 128), jnp.int32),
            pltpu.VMEM((n_fields, 128), jnp.int32),
            pltpu.VMEM((128, 128), jnp.float32),
            pltpu.VMEM((128, 128), jnp.float32),
            pltpu.VMEM((4, 8, 128), jnp.float32),
            pltpu.SemaphoreType.DMA,
            pltpu.SemaphoreType.DMA,
        ],
        compiler_params=pltpu.CompilerParams(
            use_tc_tiling_on_sc=False, needs_layout_passes=False
        ),
    )
    def k(tblq_hbm, qidx_hbm, sub_hbm, out_hbm, qidx_v, sub_v, g0, g1, ot, s0, s1):
        wid = lax.axis_index("s") * NUM_CORES + lax.axis_index("c")
        pltpu.sync_copy(qidx_hbm.at[wid], qidx_v)
        pltpu.sync_copy(sub_hbm.at[wid], sub_v)
        riota = lax.iota(jnp.int32, 16)

        def process(f, g):
            for grp in range(8):
                rows = riota + (16 * grp)
                cols = sub_v[f, pl.ds(16 * grp, 16)]
                for j in range(EMB_DIM):
                    ot[j // 8, j % 8, pl.ds(16 * grp, 16)] = plsc.load_gather(
                        g, [rows, cols + j]
                    )
            for jk in range(4):
                pltpu.sync_copy(ot.at[jk], out_hbm.at[f, jk, wid])

        pltpu.async_copy(tblq_hbm.at[qidx_v.at[0]], g0, s0)
        pltpu.async_copy(tblq_hbm.at[qidx_v.at[1]], g1, s1)

        def body(fp, carry):
            f0 = 2 * fp
            pltpu.make_async_copy(tblq_hbm.at[qidx_v.at[f0]], g0, s0).wait()
            process(f0, g0)
            pltpu.async_copy(
                tblq_hbm.at[qidx_v.at[jnp.minimum(f0 + 2, n_fields - 1)]], g0, s0
            )
            f1 = f0 + 1
            pltpu.make_async_copy(tblq_hbm.at[qidx_v.at[f1]], g1, s1).wait()
            process(f1, g1)
            pltpu.async_copy(
                tblq_hbm.at[qidx_v.at[jnp.minimum(f1 + 2, n_fields - 1)]], g1, s1
            )
            return carry

        lax.fori_loop(0, n_fields // 2, body, 0)
        pltpu.make_async_copy(tblq_hbm.at[qidx_v.at[n_fields - 1]], g0, s0).wait()
        pltpu.make_async_copy(tblq_hbm.at[qidx_v.at[n_fields - 1]], g1, s1).wait()

    return k


def kernel(x, table):
    b, f = x.shape
    xf = x.astype(jnp.int32)
    tblq = _relayout(table.T)
    q = (xf % NQ).reshape(NW, b // NW, f).transpose(0, 2, 1)
    sub = ((xf // NQ) * EMB_DIM).reshape(NW, b // NW, f).transpose(0, 2, 1)
    out5 = _make_sc_gather(f)(tblq, q, sub)
    return out5.transpose(2, 4, 0, 1, 3).reshape(b, f, EMB_DIM)
